# Initial kernel scaffold; baseline (speedup 1.0000x reference)
import jax
import jax.numpy as jnp
from jax.experimental import pallas as pl


def kernel(x, signs, indxs):
    raise NotImplementedError("write your pallas kernel here")


# speedup vs baseline: 87.2333x; 87.2333x over previous
import numpy as np
import jax
import jax.numpy as jnp
from jax.experimental import pallas as pl

_N = 1024

def _build_h() -> np.ndarray:
    i = np.arange(_N)
    parity = np.array([bin(v).count("1") & 1 for v in range(_N)], dtype=np.int8)
    return (1.0 - 2.0 * parity[i[:, None] & i[None, :]]).astype(np.float32)

_H = _build_h()


def _wht_body(x_ref, h_ref, o_ref):
    a = x_ref[0]
    h = h_ref[...]
    t = jnp.dot(h, a, preferred_element_type=jnp.float32)
    o_ref[0] = jnp.dot(t, h, preferred_element_type=jnp.float32) * (1.0 / _N)


def kernel(x, signs, indxs):
    b, dim = x.shape
    xr = x.reshape(b, _N, _N)
    out = pl.pallas_call(
        _wht_body,
        grid=(b,),
        in_specs=[
            pl.BlockSpec((1, _N, _N), lambda i: (i, 0, 0)),
            pl.BlockSpec((_N, _N), lambda i: (0, 0)),
        ],
        out_specs=pl.BlockSpec((1, _N, _N), lambda i: (i, 0, 0)),
        out_shape=jax.ShapeDtypeStruct((b, _N, _N), jnp.float32),
    )(xr, jnp.asarray(_H))
    return out.reshape(b, dim)


# speedup vs baseline: 87.8106x; 1.0066x over previous
import numpy as np
import jax
import jax.numpy as jnp
from jax.experimental import pallas as pl

_N = 1024

def _build_h() -> np.ndarray:
    i = np.arange(_N)
    parity = np.array([bin(v).count("1") & 1 for v in range(_N)], dtype=np.int8)
    return (1.0 - 2.0 * parity[i[:, None] & i[None, :]]).astype(np.float32)

_H = _build_h()


def _wht_body(x_ref, h_ref, o_ref):
    a = x_ref[0].astype(jnp.bfloat16)
    h = h_ref[...]
    t = jnp.dot(h, a, preferred_element_type=jnp.float32)
    t16 = (t * (1.0 / _N)).astype(jnp.bfloat16)
    o_ref[0] = jnp.dot(t16, h, preferred_element_type=jnp.float32)


def kernel(x, signs, indxs):
    b, dim = x.shape
    xr = x.reshape(b, _N, _N)
    out = pl.pallas_call(
        _wht_body,
        grid=(b,),
        in_specs=[
            pl.BlockSpec((1, _N, _N), lambda i: (i, 0, 0)),
            pl.BlockSpec((_N, _N), lambda i: (0, 0)),
        ],
        out_specs=pl.BlockSpec((1, _N, _N), lambda i: (i, 0, 0)),
        out_shape=jax.ShapeDtypeStruct((b, _N, _N), jnp.float32),
    )(xr, jnp.asarray(_H, dtype=jnp.bfloat16))
    return out.reshape(b, dim)


# speedup vs baseline: 157.9478x; 1.7987x over previous
import numpy as np
import jax
import jax.numpy as jnp
from jax.experimental import pallas as pl

_N = 1024
_B = 8
_SLAB = 128

def _build_h() -> np.ndarray:
    i = np.arange(_N)
    parity = np.array([bin(v).count("1") & 1 for v in range(_N)], dtype=np.int8)
    return (1.0 - 2.0 * parity[i[:, None] & i[None, :]]).astype(np.float32)

_H = _build_h()


def _right_body(x_ref, h_ref, m_ref):
    a = x_ref[...]
    a2 = a.reshape(_B, _SLAB, _N).reshape(_B * _SLAB, _N).astype(jnp.bfloat16)
    c = jnp.dot(a2, h_ref[...], preferred_element_type=jnp.float32)
    m_ref[...] = (c * (1.0 / _N)).astype(jnp.bfloat16)


def _left_body(m_ref, h_ref, o_ref):
    hs = h_ref[...]
    outs = []
    for b in range(_B):
        cb = jnp.concatenate(
            [m_ref[k2 * _N + b * _SLAB : k2 * _N + (b + 1) * _SLAB, :]
             for k2 in range(_B)], axis=0)
        outs.append(jnp.dot(hs, cb, preferred_element_type=jnp.float32))
    st = jnp.stack(outs, axis=0)
    o_ref[...] = st.reshape(_B, _SLAB * _N)


def kernel(x, signs, indxs):
    b, dim = x.shape
    h16 = jnp.asarray(_H, dtype=jnp.bfloat16)
    m = pl.pallas_call(
        _right_body,
        grid=(8,),
        in_specs=[
            pl.BlockSpec((_B, _SLAB * _N), lambda i: (0, i)),
            pl.BlockSpec((_N, _N), lambda i: (0, 0)),
        ],
        out_specs=pl.BlockSpec((_N, _N), lambda i: (i, 0)),
        out_shape=jax.ShapeDtypeStruct((_B * _N, _N), jnp.bfloat16),
    )(x, h16)
    out = pl.pallas_call(
        _left_body,
        grid=(8,),
        in_specs=[
            pl.BlockSpec((_B * _N, _N), lambda i: (0, 0)),
            pl.BlockSpec((_SLAB, _N), lambda i: (i, 0)),
        ],
        out_specs=pl.BlockSpec((_B, _SLAB * _N), lambda i: (0, i)),
        out_shape=jax.ShapeDtypeStruct((b, dim), jnp.float32),
    )(m, h16)
    return out


# speedup vs baseline: 179.0619x; 1.1337x over previous
import numpy as np
import jax
import jax.numpy as jnp
from jax.experimental import pallas as pl
from jax.experimental.pallas import tpu as pltpu

_N = 1024
_B = 8
_SLAB = 128

def _build_h() -> np.ndarray:
    i = np.arange(_N)
    parity = np.array([bin(v).count("1") & 1 for v in range(_N)], dtype=np.int8)
    return (1.0 - 2.0 * parity[i[:, None] & i[None, :]]).astype(np.float32)

_H = _build_h()


def _fused_body(x_ref, h_ref, o_ref, m_ref):
    i = pl.program_id(0)

    @pl.when(i < _B)
    def _right():
        a = x_ref[...]
        a2 = (a.reshape(_B, _SLAB, _N)
               .reshape(_B * _SLAB, _N).astype(jnp.bfloat16))
        c = jnp.dot(a2, h_ref[...], preferred_element_type=jnp.float32)
        m_ref[pl.ds(i * _N, _N), :] = (c * (1.0 / _N)).astype(jnp.bfloat16)

    @pl.when(i >= _B)
    def _left():
        k = i - _B
        hs = h_ref[pl.ds(k * _SLAB, _SLAB), :]
        outs = []
        for b in range(_B):
            cb = jnp.concatenate(
                [m_ref[k2 * _N + b * _SLAB : k2 * _N + (b + 1) * _SLAB, :]
                 for k2 in range(_B)], axis=0)
            outs.append(jnp.dot(hs, cb, preferred_element_type=jnp.float32))
        st = jnp.stack(outs, axis=0)
        o_ref[...] = st.reshape(_B, _SLAB * _N)


def kernel(x, signs, indxs):
    b, dim = x.shape
    h16 = jnp.asarray(_H, dtype=jnp.bfloat16)
    out = pl.pallas_call(
        _fused_body,
        grid=(2 * _B,),
        in_specs=[
            pl.BlockSpec((_B, _SLAB * _N), lambda i: (0, jnp.minimum(i, _B - 1))),
            pl.BlockSpec((_N, _N), lambda i: (0, 0)),
        ],
        out_specs=pl.BlockSpec((_B, _SLAB * _N),
                               lambda i: (0, jnp.maximum(i - _B, 0))),
        out_shape=jax.ShapeDtypeStruct((b, dim), jnp.float32),
        scratch_shapes=[pltpu.VMEM((_B * _N, _N), jnp.bfloat16)],
    )(x, h16)
    return out


# speedup vs baseline: 255.9271x; 1.4293x over previous
import numpy as np
import jax
import jax.numpy as jnp
from jax.experimental import pallas as pl
from jax.experimental.pallas import tpu as pltpu

_N = 1024
_B = 8
_SLAB = 128
_Q = 256

def _build_h(n: int) -> np.ndarray:
    i = np.arange(n)
    parity = np.array([bin(v).count("1") & 1 for v in range(n)], dtype=np.int8)
    return (1.0 - 2.0 * parity[i[:, None] & i[None, :]]).astype(np.float32)

_H256 = _build_h(_Q)


def _fused_body(x_ref, h_ref, o_ref, m_ref):
    i = pl.program_id(0)
    h256 = h_ref[...]

    @pl.when(i < _B)
    def _right():
        a = x_ref[...]
        a2 = (a.reshape(_B, _SLAB, _N)
               .reshape(_B * _SLAB, _N).astype(jnp.bfloat16))
        lo, hi = a2[:, :512], a2[:, 512:]
        v0, v1 = lo + hi, lo - hi
        w = [v0[:, :_Q] + v0[:, _Q:], v0[:, :_Q] - v0[:, _Q:],
             v1[:, :_Q] + v1[:, _Q:], v1[:, :_Q] - v1[:, _Q:]]
        c = jnp.concatenate(
            [jnp.dot(wb, h256, preferred_element_type=jnp.float32)
             for wb in w], axis=1)
        m_ref[pl.ds(i * _N, _N), :] = (c * (1.0 / _N)).astype(jnp.bfloat16)

    @pl.when(jnp.logical_and(i >= _B, i < _B + 2))
    def _left_h4():
        p = i - _B
        rows = [m_ref[pl.ds((2 * rh + p) * _N, _N), :] for rh in range(4)]
        t0, t1 = rows[0] + rows[2], rows[1] + rows[3]
        t2, t3 = rows[0] - rows[2], rows[1] - rows[3]
        u = [t0 + t1, t0 - t1, t2 + t3, t2 - t3]
        for rh in range(4):
            m_ref[pl.ds((2 * rh + p) * _N, _N), :] = u[rh]

    @pl.when(i >= _B + 2)
    def _left():
        k = i - (_B + 2)
        rh = k // 2
        hs = jnp.where((k % 2) == 0, h256[:_SLAB, :], h256[_SLAB:, :])
        outs = []
        for b in range(_B):
            cb = jnp.concatenate(
                [m_ref[pl.ds((2 * rh + j) * _N + b * _SLAB, _SLAB), :]
                 for j in range(2)], axis=0)
            outs.append(jnp.dot(hs, cb, preferred_element_type=jnp.float32))
        st = jnp.stack(outs, axis=0)
        o_ref[...] = st.reshape(_B, _SLAB * _N)


def kernel(x, signs, indxs):
    b, dim = x.shape
    h16 = jnp.asarray(_H256, dtype=jnp.bfloat16)
    out = pl.pallas_call(
        _fused_body,
        grid=(2 * _B + 2,),
        in_specs=[
            pl.BlockSpec((_B, _SLAB * _N), lambda i: (0, jnp.minimum(i, _B - 1))),
            pl.BlockSpec((_Q, _Q), lambda i: (0, 0)),
        ],
        out_specs=pl.BlockSpec((_B, _SLAB * _N),
                               lambda i: (0, jnp.maximum(i - (_B + 2), 0))),
        out_shape=jax.ShapeDtypeStruct((b, dim), jnp.float32),
        scratch_shapes=[pltpu.VMEM((_B * _N, _N), jnp.bfloat16)],
    )(x, h16)
    return out
